# baseline (speedup 1.0000x reference)
import functools

import jax
import jax.numpy as jnp
from jax import lax
from jax.experimental import pallas as pl
from jax.experimental.pallas import tpu as pltpu
from jax.experimental.pallas import tpu_sc as plsc

N = 10000
D = 128
H = 64
C = 16
EPS = 0.3

NPAD = 10240
DUMMY = NPAD - 1
NW = 32
CHUNK = 128
ROWS_PER_TILE = NPAD // NW * 2

_f32 = jnp.float32


def _mesh():
    return plsc.VectorSubcoreMesh(core_axis_name="c", subcore_axis_name="s")


_SC_PARAMS = pltpu.CompilerParams(
    needs_layout_passes=False, use_tc_tiling_on_sc=False)


def _sc_deg(pk3d, ch):
    @functools.partial(
        pl.kernel,
        out_type=jax.ShapeDtypeStruct((2, NPAD), _f32),
        mesh=_mesh(),
        compiler_params=_SC_PARAMS,
        scratch_types=[
            pltpu.VMEM((ch, CHUNK), jnp.int32),
            pltpu.VMEM((CHUNK,), jnp.int32),
            pltpu.VMEM((CHUNK,), _f32),
            pltpu.VMEM((ROWS_PER_TILE,), _f32),
            pltpu.VMEM_SHARED((NPAD,), _f32),
            pltpu.SemaphoreType.DMA,
        ],
    )
    def k(pk_hbm, out_hbm, pki, ridx, ones_v, z_v, deg_sh, sem):
        c = lax.axis_index("c")
        s = lax.axis_index("s")
        wid = c * 16 + s
        for i in range(CHUNK // 16):
            ones_v[pl.ds(i * 16, 16)] = jnp.full((16,), 1.0, _f32)
        for i in range(ROWS_PER_TILE // 16):
            z_v[pl.ds(i * 16, 16)] = jnp.zeros((16,), _f32)
        pltpu.sync_copy(z_v, deg_sh.at[pl.ds(s * ROWS_PER_TILE, ROWS_PER_TILE)])
        pltpu.sync_copy(pk_hbm.at[wid], pki)
        plsc.subcore_barrier()

        def body(j, carry):
            for i in range(CHUNK // 16):
                p16 = pki[j, pl.ds(i * 16, 16)]
                ridx[pl.ds(i * 16, 16)] = jnp.bitwise_and(p16, 0xFFFF)
            pltpu.sync_copy(ones_v, deg_sh.at[ridx], add=True)
            return carry

        lax.fori_loop(0, ch, body, 0)
        plsc.subcore_barrier()
        pltpu.sync_copy(
            deg_sh.at[pl.ds(s * ROWS_PER_TILE, ROWS_PER_TILE)],
            out_hbm.at[c, pl.ds(s * ROWS_PER_TILE, ROWS_PER_TILE)],
        )

    return k(pk3d)


def _sc_edge(hh, S, pk3d, zeros_big, ch):
    @functools.partial(
        pl.kernel,
        out_type=jax.ShapeDtypeStruct((2, NPAD, H), _f32),
        mesh=_mesh(),
        compiler_params=_SC_PARAMS,
        scratch_types=[
            pltpu.VMEM((ch, CHUNK), jnp.int32),
            pltpu.VMEM((NPAD,), _f32),
            pltpu.VMEM((NPAD,), _f32),
            pltpu.VMEM((4, CHUNK), jnp.int32),
            pltpu.VMEM((2, CHUNK), jnp.int32),
            pltpu.VMEM((CHUNK, H), _f32),
            pltpu.VMEM((CHUNK, H), _f32),
            pltpu.VMEM((CHUNK, H), _f32),
            pltpu.VMEM((CHUNK, H), _f32),
            pltpu.VMEM((CHUNK, H), _f32),
            pltpu.VMEM((CHUNK, H), _f32),
            pltpu.VMEM_SHARED((NPAD, H), _f32),
            pltpu.SemaphoreType.DMA,
            pltpu.SemaphoreType.DMA,
            pltpu.SemaphoreType.DMA,
            pltpu.SemaphoreType.DMA,
            pltpu.SemaphoreType.DMA,
            pltpu.SemaphoreType.DMA,
        ],
    )
    def k(hh_hbm, s_hbm, pk_hbm, z_hbm, out_hbm,
          pki, atab, btab, ridx, cidx, g0, g1, g2, g3, s0, s1, agg_sh,
          gsem0, gsem1, gsem2, gsem3, ssem0, ssem1):
        c = lax.axis_index("c")
        s = lax.axis_index("s")
        wid = c * 16 + s
        gbuf = (g0, g1, g2, g3)
        sbuf = (s0, s1)
        gsem = (gsem0, gsem1, gsem2, gsem3)
        ssem = (ssem0, ssem1)
        pltpu.sync_copy(s_hbm.at[0], atab)
        pltpu.sync_copy(s_hbm.at[1], btab)
        pltpu.sync_copy(pk_hbm.at[wid], pki)
        pltpu.sync_copy(
            z_hbm.at[pl.ds(s * ROWS_PER_TILE, ROWS_PER_TILE)],
            agg_sh.at[pl.ds(s * ROWS_PER_TILE, ROWS_PER_TILE)],
        )
        plsc.subcore_barrier()

        def fill_ridx(j, slot):
            for i in range(CHUNK // 16):
                p16 = pki[j, pl.ds(i * 16, 16)]
                ridx[slot, pl.ds(i * 16, 16)] = jnp.bitwise_and(p16, 0xFFFF)

        for b in range(4):
            fill_ridx(b, b)
            pltpu.async_copy(hh_hbm.at[ridx.at[b]], gbuf[b], gsem[b])
        ch4 = ch // 4

        def quad(jj, carry):
            for b in range(4):
                j = 4 * jj + b
                sb = b % 2
                pltpu.make_async_copy(
                    hh_hbm.at[ridx.at[b]], gbuf[b], gsem[b]).wait()
                if b >= 2:
                    pltpu.make_async_copy(
                        sbuf[sb], agg_sh.at[cidx.at[sb]], ssem[sb]).wait()
                else:
                    @pl.when(jj >= 1)
                    def _():
                        pltpu.make_async_copy(
                            sbuf[sb], agg_sh.at[cidx.at[sb]], ssem[sb]).wait()
                for i in range(CHUNK // 16):
                    p16 = pki[j, pl.ds(i * 16, 16)]
                    r16 = jnp.bitwise_and(p16, 0xFFFF)
                    c16 = lax.shift_right_logical(p16, 16)
                    cidx[sb, pl.ds(i * 16, 16)] = c16
                    ag = plsc.load_gather(atab, [r16])
                    bg = plsc.load_gather(btab, [c16])
                    t = ag + bg
                    sg = jnp.sign(t)
                    u = jnp.exp(-2.0 * jnp.abs(t))
                    nv = sg * (1.0 - u) / (1.0 + u)
                    for e in range(16):
                        be = jnp.take_along_axis(
                            nv, jnp.full((16,), e, jnp.int32), axis=0)
                        r = i * 16 + e
                        for q in range(H // 16):
                            sbuf[sb][r, pl.ds(q * 16, 16)] = (
                                gbuf[b][r, pl.ds(q * 16, 16)] * be)
                pltpu.async_copy(
                    sbuf[sb], agg_sh.at[cidx.at[sb]], ssem[sb], add=True)
                @pl.when(jj < ch4 - 1)
                def _():
                    fill_ridx(j + 4, b)
                    pltpu.async_copy(
                        hh_hbm.at[ridx.at[b]], gbuf[b], gsem[b])
            return carry

        lax.fori_loop(0, ch4, quad, 0)
        for sb in range(2):
            pltpu.make_async_copy(
                sbuf[sb], agg_sh.at[cidx.at[sb]], ssem[sb]).wait()
        plsc.subcore_barrier()
        pltpu.sync_copy(
            agg_sh.at[pl.ds(s * ROWS_PER_TILE, ROWS_PER_TILE)],
            out_hbm.at[c, pl.ds(s * ROWS_PER_TILE, ROWS_PER_TILE)],
        )

    return k(hh, S, pk3d, zeros_big)


_DN = (((1,), (1,)), ((), ()))
_PREC = lax.Precision.HIGHEST
BT = 2048


def _tc_pre(x_pad, t1_W, t1_b2, G8, gb, deg2):
    def body(x_ref, w_ref, b_ref, g_ref, gb_ref, d_ref, h_ref, hh_ref, s_ref):
        xb = x_ref[...]
        hv = lax.dot_general(xb, w_ref[...], _DN, precision=_PREC) + b_ref[...]
        hv = jnp.maximum(hv, 0.0)
        h_ref[...] = hv
        deg = jnp.maximum(d_ref[0, :] + d_ref[1, :], 1.0)
        nd = lax.rsqrt(deg)
        hh_ref[...] = hv * nd[:, None]
        sdot = lax.dot_general(g_ref[...], hv, _DN, precision=_PREC)
        ri = lax.broadcasted_iota(jnp.int32, (8, BT), 0)
        s_ref[...] = sdot + jnp.where(ri == 0, gb_ref[0, 0], 0.0)

    return pl.pallas_call(
        body,
        grid=(NPAD // BT,),
        in_specs=[
            pl.BlockSpec((BT, D), lambda i: (i, 0)),
            pl.BlockSpec((H, D), lambda i: (0, 0)),
            pl.BlockSpec((1, H), lambda i: (0, 0)),
            pl.BlockSpec((8, H), lambda i: (0, 0)),
            pl.BlockSpec((1, 1), lambda i: (0, 0)),
            pl.BlockSpec((2, BT), lambda i: (0, i)),
        ],
        out_specs=[
            pl.BlockSpec((BT, H), lambda i: (i, 0)),
            pl.BlockSpec((BT, H), lambda i: (i, 0)),
            pl.BlockSpec((8, BT), lambda i: (0, i)),
        ],
        out_shape=[
            jax.ShapeDtypeStruct((NPAD, H), _f32),
            jax.ShapeDtypeStruct((NPAD, H), _f32),
            jax.ShapeDtypeStruct((8, NPAD), _f32),
        ],
    )(x_pad, t1_W, t1_b2, G8, gb, deg2)


def _tc_mid(agg, h0, G8, gb, deg2):
    def body(a_ref, h0_ref, g_ref, gb_ref, d_ref, hh_ref, s_ref):
        deg = jnp.maximum(d_ref[0, :] + d_ref[1, :], 1.0)
        nd = lax.rsqrt(deg)
        hv = EPS * h0_ref[...] + (a_ref[0] + a_ref[1]) * nd[:, None]
        hh_ref[...] = hv * nd[:, None]
        sdot = lax.dot_general(g_ref[...], hv, _DN, precision=_PREC)
        ri = lax.broadcasted_iota(jnp.int32, (8, BT), 0)
        s_ref[...] = sdot + jnp.where(ri == 0, gb_ref[0, 0], 0.0)

    return pl.pallas_call(
        body,
        grid=(NPAD // BT,),
        in_specs=[
            pl.BlockSpec((2, BT, H), lambda i: (0, i, 0)),
            pl.BlockSpec((BT, H), lambda i: (i, 0)),
            pl.BlockSpec((8, H), lambda i: (0, 0)),
            pl.BlockSpec((1, 1), lambda i: (0, 0)),
            pl.BlockSpec((2, BT), lambda i: (0, i)),
        ],
        out_specs=[
            pl.BlockSpec((BT, H), lambda i: (i, 0)),
            pl.BlockSpec((8, BT), lambda i: (0, i)),
        ],
        out_shape=[
            jax.ShapeDtypeStruct((NPAD, H), _f32),
            jax.ShapeDtypeStruct((8, NPAD), _f32),
        ],
    )(agg, h0, G8, gb, deg2)


BF = 2000


def _tc_final(agg, h0, t2_W, t2_b2, deg2t):
    def body(a_ref, h0_ref, w_ref, b_ref, d_ref, o_ref):
        deg = jnp.maximum(d_ref[:, 0] + d_ref[:, 1], 1.0)
        nd = lax.rsqrt(deg)
        hv = EPS * h0_ref[...] + (a_ref[0] + a_ref[1]) * nd[:, None]
        o = lax.dot_general(hv, w_ref[...], _DN, precision=_PREC) + b_ref[...]
        m = jnp.max(o, axis=1, keepdims=True)
        z = o - m
        lse = jnp.log(jnp.sum(jnp.exp(z), axis=1, keepdims=True))
        o_ref[...] = z - lse

    return pl.pallas_call(
        body,
        grid=(N // BF,),
        in_specs=[
            pl.BlockSpec((2, BF, H), lambda i: (0, i, 0)),
            pl.BlockSpec((BF, H), lambda i: (i, 0)),
            pl.BlockSpec((C, H), lambda i: (0, 0)),
            pl.BlockSpec((1, C), lambda i: (0, 0)),
            pl.BlockSpec((BF, 2), lambda i: (i, 0)),
        ],
        out_specs=pl.BlockSpec((BF, C), lambda i: (i, 0)),
        out_shape=jax.ShapeDtypeStruct((N, C), _f32),
    )(agg, h0, t2_W, t2_b2, deg2t)


def kernel(x, edge_index, t1_W, t1_b, t2_W, t2_b, gate_W, gate_b):
    E = edge_index.shape[1]
    ept = -(-E // NW)
    ch = -(-ept // CHUNK)
    ch = -(-ch // 4) * 4
    EP = NW * ch * CHUNK

    row = jnp.pad(edge_index[0], (0, EP - E), constant_values=DUMMY)
    col = jnp.pad(edge_index[1], (0, EP - E), constant_values=DUMMY)
    pk3d = jnp.bitwise_or(row, jnp.left_shift(col, 16)).reshape(NW, ch, CHUNK)

    x_pad = jnp.pad(x, ((0, NPAD - x.shape[0]), (0, 0)))
    t1_b2 = t1_b.reshape(1, H)
    t2_b2 = t2_b.reshape(1, C)
    G80 = jnp.zeros((8, H), _f32).at[0].set(gate_W[0, :H]).at[1].set(gate_W[0, H:])
    G81 = jnp.zeros((8, H), _f32).at[0].set(gate_W[1, :H]).at[1].set(gate_W[1, H:])
    gb0 = gate_b[0].reshape(1, 1)
    gb1 = gate_b[1].reshape(1, 1)
    zeros_big = jnp.zeros((NPAD, H), _f32)

    deg2 = _sc_deg(pk3d, ch)
    h0, hh0, S0 = _tc_pre(x_pad, t1_W, t1_b2, G80, gb0, deg2)
    agg = _sc_edge(hh0, S0, pk3d, zeros_big, ch)
    hh1, S1 = _tc_mid(agg, h0, G81, gb1, deg2)
    agg2 = _sc_edge(hh1, S1, pk3d, zeros_big, ch)
    return _tc_final(agg2, h0, t2_W, t2_b2, deg2.T)


# speedup vs baseline: 1.1172x; 1.1172x over previous
import functools

import jax
import jax.numpy as jnp
from jax import lax
from jax.experimental import pallas as pl
from jax.experimental.pallas import tpu as pltpu
from jax.experimental.pallas import tpu_sc as plsc

N = 10000
D = 128
H = 64
C = 16
EPS = 0.3

NPAD = 10240
DUMMY = NPAD - 1
NW = 32
CHUNK = 128
ROWS_PER_TILE = NPAD // NW * 2

_f32 = jnp.float32


def _mesh():
    return plsc.VectorSubcoreMesh(core_axis_name="c", subcore_axis_name="s")


_SC_PARAMS = pltpu.CompilerParams(
    needs_layout_passes=False, use_tc_tiling_on_sc=False)


def _sc_deg(row3d, ch):
    @functools.partial(
        pl.kernel,
        out_type=jax.ShapeDtypeStruct((2, NPAD), _f32),
        mesh=_mesh(),
        compiler_params=_SC_PARAMS,
        scratch_types=[
            pltpu.VMEM((ch, CHUNK), jnp.int32),
            pltpu.VMEM((CHUNK,), _f32),
            pltpu.VMEM((ROWS_PER_TILE,), _f32),
            pltpu.VMEM_SHARED((NPAD,), _f32),
            pltpu.SemaphoreType.DMA,
        ],
    )
    def k(row_hbm, out_hbm, idx_v, ones_v, z_v, deg_sh, sem):
        c = lax.axis_index("c")
        s = lax.axis_index("s")
        wid = c * 16 + s
        for i in range(CHUNK // 16):
            ones_v[pl.ds(i * 16, 16)] = jnp.full((16,), 1.0, _f32)
        for i in range(ROWS_PER_TILE // 16):
            z_v[pl.ds(i * 16, 16)] = jnp.zeros((16,), _f32)
        pltpu.sync_copy(z_v, deg_sh.at[pl.ds(s * ROWS_PER_TILE, ROWS_PER_TILE)])
        pltpu.sync_copy(row_hbm.at[wid], idx_v)
        plsc.subcore_barrier()

        def body(j, carry):
            pltpu.sync_copy(ones_v, deg_sh.at[idx_v.at[j]], add=True)
            return carry

        lax.fori_loop(0, ch, body, 0)
        plsc.subcore_barrier()
        pltpu.sync_copy(
            deg_sh.at[pl.ds(s * ROWS_PER_TILE, ROWS_PER_TILE)],
            out_hbm.at[c, pl.ds(s * ROWS_PER_TILE, ROWS_PER_TILE)],
        )

    return k(row3d)


def _sc_edge(hh, S, row3d, col3d, zeros_big, ch):
    @functools.partial(
        pl.kernel,
        out_type=jax.ShapeDtypeStruct((2, NPAD, H), _f32),
        mesh=_mesh(),
        compiler_params=_SC_PARAMS,
        scratch_types=[
            pltpu.VMEM((ch, CHUNK), jnp.int32),
            pltpu.VMEM((ch, CHUNK), jnp.int32),
            pltpu.VMEM((NPAD,), _f32),
            pltpu.VMEM((NPAD,), _f32),
            pltpu.VMEM((CHUNK, H), _f32),
            pltpu.VMEM((CHUNK, H), _f32),
            pltpu.VMEM((CHUNK, H), _f32),
            pltpu.VMEM((CHUNK, H), _f32),
            pltpu.VMEM_SHARED((NPAD, H), _f32),
            pltpu.SemaphoreType.DMA,
            pltpu.SemaphoreType.DMA,
            pltpu.SemaphoreType.DMA,
            pltpu.SemaphoreType.DMA,
        ],
    )
    def k(hh_hbm, s_hbm, row_hbm, col_hbm, z_hbm, out_hbm,
          rowi, coli, atab, btab, g0, g1, s0, s1, agg_sh,
          gsem0, gsem1, ssem0, ssem1):
        c = lax.axis_index("c")
        s = lax.axis_index("s")
        wid = c * 16 + s
        gbuf = (g0, g1)
        sbuf = (s0, s1)
        gsem = (gsem0, gsem1)
        ssem = (ssem0, ssem1)
        pltpu.sync_copy(s_hbm.at[0], atab)
        pltpu.sync_copy(s_hbm.at[1], btab)
        pltpu.sync_copy(row_hbm.at[wid], rowi)
        pltpu.sync_copy(col_hbm.at[wid], coli)
        pltpu.sync_copy(
            z_hbm.at[pl.ds(s * ROWS_PER_TILE, ROWS_PER_TILE)],
            agg_sh.at[pl.ds(s * ROWS_PER_TILE, ROWS_PER_TILE)],
        )
        plsc.subcore_barrier()

        pltpu.async_copy(hh_hbm.at[rowi.at[0]], g0, gsem0)

        def pair(jj, carry):
            for b in range(2):
                j = 2 * jj + b
                @pl.when(j + 1 < ch)
                def _():
                    pltpu.async_copy(
                        hh_hbm.at[rowi.at[j + 1]], gbuf[1 - b], gsem[1 - b])
                pltpu.make_async_copy(
                    hh_hbm.at[rowi.at[j]], gbuf[b], gsem[b]).wait()
                @pl.when(jj >= 1)
                def _():
                    pltpu.make_async_copy(
                        sbuf[b], agg_sh.at[coli.at[j]], ssem[b]).wait()
                for i in range(CHUNK // 16):
                    r16 = rowi[j, pl.ds(i * 16, 16)]
                    c16 = coli[j, pl.ds(i * 16, 16)]
                    ag = plsc.load_gather(atab, [r16])
                    bg = plsc.load_gather(btab, [c16])
                    t = ag + bg
                    sg = jnp.sign(t)
                    u = jnp.exp(-2.0 * jnp.abs(t))
                    nv = sg * (1.0 - u) / (1.0 + u)
                    for e in range(16):
                        be = jnp.take_along_axis(
                            nv, jnp.full((16,), e, jnp.int32), axis=0)
                        r = i * 16 + e
                        for q in range(H // 16):
                            sbuf[b][r, pl.ds(q * 16, 16)] = (
                                gbuf[b][r, pl.ds(q * 16, 16)] * be)
                pltpu.async_copy(
                    sbuf[b], agg_sh.at[coli.at[j]], ssem[b], add=True)
            return carry

        lax.fori_loop(0, ch // 2, pair, 0)
        for b in range(2):
            pltpu.make_async_copy(
                sbuf[b], agg_sh.at[coli.at[0]], ssem[b]).wait()
        plsc.subcore_barrier()
        pltpu.sync_copy(
            agg_sh.at[pl.ds(s * ROWS_PER_TILE, ROWS_PER_TILE)],
            out_hbm.at[c, pl.ds(s * ROWS_PER_TILE, ROWS_PER_TILE)],
        )

    return k(hh, S, row3d, col3d, zeros_big)


_DN = (((1,), (1,)), ((), ()))
_PREC = lax.Precision.HIGHEST
BT = 2048


def _tc_pre(x_pad, t1_W, t1_b2, G8, gb, deg2):
    def body(x_ref, w_ref, b_ref, g_ref, gb_ref, d_ref, h_ref, hh_ref, s_ref):
        xb = x_ref[...]
        hv = lax.dot_general(xb, w_ref[...], _DN, precision=_PREC) + b_ref[...]
        hv = jnp.maximum(hv, 0.0)
        h_ref[...] = hv
        deg = jnp.maximum(d_ref[0, :] + d_ref[1, :], 1.0)
        nd = lax.rsqrt(deg)
        hh_ref[...] = hv * nd[:, None]
        sdot = lax.dot_general(g_ref[...], hv, _DN, precision=_PREC)
        ri = lax.broadcasted_iota(jnp.int32, (8, BT), 0)
        s_ref[...] = sdot + jnp.where(ri == 0, gb_ref[0, 0], 0.0)

    return pl.pallas_call(
        body,
        grid=(NPAD // BT,),
        in_specs=[
            pl.BlockSpec((BT, D), lambda i: (i, 0)),
            pl.BlockSpec((H, D), lambda i: (0, 0)),
            pl.BlockSpec((1, H), lambda i: (0, 0)),
            pl.BlockSpec((8, H), lambda i: (0, 0)),
            pl.BlockSpec((1, 1), lambda i: (0, 0)),
            pl.BlockSpec((2, BT), lambda i: (0, i)),
        ],
        out_specs=[
            pl.BlockSpec((BT, H), lambda i: (i, 0)),
            pl.BlockSpec((BT, H), lambda i: (i, 0)),
            pl.BlockSpec((8, BT), lambda i: (0, i)),
        ],
        out_shape=[
            jax.ShapeDtypeStruct((NPAD, H), _f32),
            jax.ShapeDtypeStruct((NPAD, H), _f32),
            jax.ShapeDtypeStruct((8, NPAD), _f32),
        ],
    )(x_pad, t1_W, t1_b2, G8, gb, deg2)


def _tc_mid(agg, h0, G8, gb, deg2):
    def body(a_ref, h0_ref, g_ref, gb_ref, d_ref, hh_ref, s_ref):
        deg = jnp.maximum(d_ref[0, :] + d_ref[1, :], 1.0)
        nd = lax.rsqrt(deg)
        hv = EPS * h0_ref[...] + (a_ref[0] + a_ref[1]) * nd[:, None]
        hh_ref[...] = hv * nd[:, None]
        sdot = lax.dot_general(g_ref[...], hv, _DN, precision=_PREC)
        ri = lax.broadcasted_iota(jnp.int32, (8, BT), 0)
        s_ref[...] = sdot + jnp.where(ri == 0, gb_ref[0, 0], 0.0)

    return pl.pallas_call(
        body,
        grid=(NPAD // BT,),
        in_specs=[
            pl.BlockSpec((2, BT, H), lambda i: (0, i, 0)),
            pl.BlockSpec((BT, H), lambda i: (i, 0)),
            pl.BlockSpec((8, H), lambda i: (0, 0)),
            pl.BlockSpec((1, 1), lambda i: (0, 0)),
            pl.BlockSpec((2, BT), lambda i: (0, i)),
        ],
        out_specs=[
            pl.BlockSpec((BT, H), lambda i: (i, 0)),
            pl.BlockSpec((8, BT), lambda i: (0, i)),
        ],
        out_shape=[
            jax.ShapeDtypeStruct((NPAD, H), _f32),
            jax.ShapeDtypeStruct((8, NPAD), _f32),
        ],
    )(agg, h0, G8, gb, deg2)


BF = 2000


def _tc_final(agg, h0, t2_W, t2_b2, deg2t):
    def body(a_ref, h0_ref, w_ref, b_ref, d_ref, o_ref):
        deg = jnp.maximum(d_ref[:, 0] + d_ref[:, 1], 1.0)
        nd = lax.rsqrt(deg)
        hv = EPS * h0_ref[...] + (a_ref[0] + a_ref[1]) * nd[:, None]
        o = lax.dot_general(hv, w_ref[...], _DN, precision=_PREC) + b_ref[...]
        m = jnp.max(o, axis=1, keepdims=True)
        z = o - m
        lse = jnp.log(jnp.sum(jnp.exp(z), axis=1, keepdims=True))
        o_ref[...] = z - lse

    return pl.pallas_call(
        body,
        grid=(N // BF,),
        in_specs=[
            pl.BlockSpec((2, BF, H), lambda i: (0, i, 0)),
            pl.BlockSpec((BF, H), lambda i: (i, 0)),
            pl.BlockSpec((C, H), lambda i: (0, 0)),
            pl.BlockSpec((1, C), lambda i: (0, 0)),
            pl.BlockSpec((BF, 2), lambda i: (i, 0)),
        ],
        out_specs=pl.BlockSpec((BF, C), lambda i: (i, 0)),
        out_shape=jax.ShapeDtypeStruct((N, C), _f32),
    )(agg, h0, t2_W, t2_b2, deg2t)


def kernel(x, edge_index, t1_W, t1_b, t2_W, t2_b, gate_W, gate_b):
    E = edge_index.shape[1]
    ept = -(-E // NW)
    ch = -(-ept // CHUNK)
    ch = -(-ch // 4) * 4
    EP = NW * ch * CHUNK

    row = jnp.pad(edge_index[0], (0, EP - E), constant_values=DUMMY)
    col = jnp.pad(edge_index[1], (0, EP - E), constant_values=DUMMY)
    row3d = row.reshape(NW, ch, CHUNK)
    col3d = col.reshape(NW, ch, CHUNK)

    x_pad = jnp.pad(x, ((0, NPAD - x.shape[0]), (0, 0)))
    t1_b2 = t1_b.reshape(1, H)
    t2_b2 = t2_b.reshape(1, C)
    G80 = jnp.zeros((8, H), _f32).at[0].set(gate_W[0, :H]).at[1].set(gate_W[0, H:])
    G81 = jnp.zeros((8, H), _f32).at[0].set(gate_W[1, :H]).at[1].set(gate_W[1, H:])
    gb0 = gate_b[0].reshape(1, 1)
    gb1 = gate_b[1].reshape(1, 1)
    zeros_big = jnp.zeros((NPAD, H), _f32)

    deg2 = _sc_deg(row3d, ch)
    h0, hh0, S0 = _tc_pre(x_pad, t1_W, t1_b2, G80, gb0, deg2)
    agg = _sc_edge(hh0, S0, row3d, col3d, zeros_big, ch)
    hh1, S1 = _tc_mid(agg, h0, G81, gb1, deg2)
    agg2 = _sc_edge(hh1, S1, row3d, col3d, zeros_big, ch)
    return _tc_final(agg2, h0, t2_W, t2_b2, deg2.T)


# speedup vs baseline: 1.6080x; 1.4393x over previous
import functools

import jax
import jax.numpy as jnp
import numpy as np
from jax import lax
from jax.experimental import pallas as pl
from jax.experimental.pallas import tpu as pltpu
from jax.experimental.pallas import tpu_sc as plsc

N = 10000
D = 128
H = 64
C = 16
EPS = 0.3

NPAD = 10240
DUMMY = NPAD - 1
NW = 32
CHUNK = 128
ROWS_PER_TILE = NPAD // NW * 2

_f32 = jnp.float32


def _mesh():
    return plsc.VectorSubcoreMesh(core_axis_name="c", subcore_axis_name="s")


_SC_PARAMS = pltpu.CompilerParams(
    needs_layout_passes=False, use_tc_tiling_on_sc=False)

_PERM = np.zeros((H,), np.int32)
for _q in range(H // 32):
    for _t in range(16):
        _PERM[32 * _q + 2 * _t] = 32 * _q + _t
        _PERM[32 * _q + 2 * _t + 1] = 32 * _q + 16 + _t
_BF16 = jnp.bfloat16


def _sc_deg(row3d, ch):
    @functools.partial(
        pl.kernel,
        out_type=jax.ShapeDtypeStruct((2, NPAD), _f32),
        mesh=_mesh(),
        compiler_params=_SC_PARAMS,
        scratch_types=[
            pltpu.VMEM((ch, CHUNK), jnp.int32),
            pltpu.VMEM((CHUNK,), _f32),
            pltpu.VMEM((ROWS_PER_TILE,), _f32),
            pltpu.VMEM_SHARED((NPAD,), _f32),
            pltpu.SemaphoreType.DMA,
        ],
    )
    def k(row_hbm, out_hbm, idx_v, ones_v, z_v, deg_sh, sem):
        c = lax.axis_index("c")
        s = lax.axis_index("s")
        wid = c * 16 + s
        for i in range(CHUNK // 16):
            ones_v[pl.ds(i * 16, 16)] = jnp.full((16,), 1.0, _f32)
        for i in range(ROWS_PER_TILE // 16):
            z_v[pl.ds(i * 16, 16)] = jnp.zeros((16,), _f32)
        pltpu.sync_copy(z_v, deg_sh.at[pl.ds(s * ROWS_PER_TILE, ROWS_PER_TILE)])
        pltpu.sync_copy(row_hbm.at[wid], idx_v)
        plsc.subcore_barrier()

        def body(j, carry):
            pltpu.sync_copy(ones_v, deg_sh.at[idx_v.at[j]], add=True)
            return carry

        lax.fori_loop(0, ch, body, 0)
        plsc.subcore_barrier()
        pltpu.sync_copy(
            deg_sh.at[pl.ds(s * ROWS_PER_TILE, ROWS_PER_TILE)],
            out_hbm.at[c, pl.ds(s * ROWS_PER_TILE, ROWS_PER_TILE)],
        )

    return k(row3d)


def _sc_edge(hh, S, row3d, col3d, zeros_big, ch):
    @functools.partial(
        pl.kernel,
        out_type=jax.ShapeDtypeStruct((2, NPAD, H), _f32),
        mesh=_mesh(),
        compiler_params=_SC_PARAMS,
        scratch_types=[
            pltpu.VMEM((ch, CHUNK), jnp.int32),
            pltpu.VMEM((ch, CHUNK), jnp.int32),
            pltpu.VMEM((NPAD,), _f32),
            pltpu.VMEM((NPAD,), _f32),
            pltpu.VMEM((CHUNK, H), _BF16),
            pltpu.VMEM((CHUNK, H), _BF16),
            pltpu.VMEM((CHUNK, H), _f32),
            pltpu.VMEM((CHUNK, H), _f32),
            pltpu.VMEM_SHARED((NPAD, H), _f32),
            pltpu.SemaphoreType.DMA,
            pltpu.SemaphoreType.DMA,
            pltpu.SemaphoreType.DMA,
            pltpu.SemaphoreType.DMA,
        ],
    )
    def k(hh_hbm, s_hbm, row_hbm, col_hbm, z_hbm, out_hbm,
          rowi, coli, atab, btab, g0, g1, s0, s1, agg_sh,
          gsem0, gsem1, ssem0, ssem1):
        c = lax.axis_index("c")
        s = lax.axis_index("s")
        wid = c * 16 + s
        gbuf = (g0, g1)
        sbuf = (s0, s1)
        gsem = (gsem0, gsem1)
        ssem = (ssem0, ssem1)
        pltpu.sync_copy(s_hbm.at[0], atab)
        pltpu.sync_copy(s_hbm.at[1], btab)
        pltpu.sync_copy(row_hbm.at[wid], rowi)
        pltpu.sync_copy(col_hbm.at[wid], coli)
        pltpu.sync_copy(
            z_hbm.at[pl.ds(s * ROWS_PER_TILE, ROWS_PER_TILE)],
            agg_sh.at[pl.ds(s * ROWS_PER_TILE, ROWS_PER_TILE)],
        )
        plsc.subcore_barrier()

        pltpu.async_copy(hh_hbm.at[rowi.at[0]], g0, gsem0)

        def pair(jj, carry):
            for b in range(2):
                j = 2 * jj + b
                @pl.when(j + 1 < ch)
                def _():
                    pltpu.async_copy(
                        hh_hbm.at[rowi.at[j + 1]], gbuf[1 - b], gsem[1 - b])
                pltpu.make_async_copy(
                    hh_hbm.at[rowi.at[j]], gbuf[b], gsem[b]).wait()
                @pl.when(jj >= 1)
                def _():
                    pltpu.make_async_copy(
                        sbuf[b], agg_sh.at[coli.at[j]], ssem[b]).wait()
                for i in range(CHUNK // 16):
                    r16 = rowi[j, pl.ds(i * 16, 16)]
                    c16 = coli[j, pl.ds(i * 16, 16)]
                    ag = plsc.load_gather(atab, [r16])
                    bg = plsc.load_gather(btab, [c16])
                    t = ag + bg
                    sg = jnp.sign(t)
                    u = jnp.exp(-2.0 * jnp.abs(t))
                    nv = sg * (1.0 - u) / (1.0 + u)
                    for e in range(16):
                        be = jnp.take_along_axis(
                            nv, jnp.full((16,), e, jnp.int32), axis=0)
                        r = i * 16 + e
                        for q in range(H // 32):
                            x32 = gbuf[b][r, pl.ds(q * 32, 32)]
                            u, v = plsc.unpack(
                                x32, format=plsc.PackFormat.INTERLEAVED)
                            sbuf[b][r, pl.ds(q * 32, 16)] = u * be
                            sbuf[b][r, pl.ds(q * 32 + 16, 16)] = v * be
                pltpu.async_copy(
                    sbuf[b], agg_sh.at[coli.at[j]], ssem[b], add=True)
            return carry

        lax.fori_loop(0, ch // 2, pair, 0)
        for b in range(2):
            pltpu.make_async_copy(
                sbuf[b], agg_sh.at[coli.at[0]], ssem[b]).wait()
        plsc.subcore_barrier()
        pltpu.sync_copy(
            agg_sh.at[pl.ds(s * ROWS_PER_TILE, ROWS_PER_TILE)],
            out_hbm.at[c, pl.ds(s * ROWS_PER_TILE, ROWS_PER_TILE)],
        )

    return k(hh, S, row3d, col3d, zeros_big)


_DN = (((1,), (1,)), ((), ()))
_PREC = lax.Precision.HIGHEST
BT = 2048


def _tc_pre(x_pad, t1_W, t1_b2, G8, gb, deg2, perm):
    def body(x_ref, w_ref, b_ref, g_ref, gb_ref, d_ref, p_ref,
             h_ref, hh_ref, s_ref):
        xb = x_ref[...]
        hv = lax.dot_general(xb, w_ref[...], _DN, precision=_PREC) + b_ref[...]
        hv = jnp.maximum(hv, 0.0)
        h_ref[...] = hv
        deg = jnp.maximum(d_ref[0, :] + d_ref[1, :], 1.0)
        nd = lax.rsqrt(deg)
        hhv = hv * nd[:, None]
        pidx = jnp.broadcast_to(p_ref[...], (BT, H))
        hh_ref[...] = jnp.take_along_axis(hhv, pidx, axis=1).astype(_BF16)
        sdot = lax.dot_general(g_ref[...], hv, _DN, precision=_PREC)
        ri = lax.broadcasted_iota(jnp.int32, (8, BT), 0)
        s_ref[...] = sdot + jnp.where(ri == 0, gb_ref[0, 0], 0.0)

    return pl.pallas_call(
        body,
        grid=(NPAD // BT,),
        in_specs=[
            pl.BlockSpec((BT, D), lambda i: (i, 0)),
            pl.BlockSpec((H, D), lambda i: (0, 0)),
            pl.BlockSpec((1, H), lambda i: (0, 0)),
            pl.BlockSpec((8, H), lambda i: (0, 0)),
            pl.BlockSpec((1, 1), lambda i: (0, 0)),
            pl.BlockSpec((2, BT), lambda i: (0, i)),
            pl.BlockSpec((1, H), lambda i: (0, 0)),
        ],
        out_specs=[
            pl.BlockSpec((BT, H), lambda i: (i, 0)),
            pl.BlockSpec((BT, H), lambda i: (i, 0)),
            pl.BlockSpec((8, BT), lambda i: (0, i)),
        ],
        out_shape=[
            jax.ShapeDtypeStruct((NPAD, H), _f32),
            jax.ShapeDtypeStruct((NPAD, H), _BF16),
            jax.ShapeDtypeStruct((8, NPAD), _f32),
        ],
    )(x_pad, t1_W, t1_b2, G8, gb, deg2, perm)


def _tc_mid(agg, h0, G8, gb, deg2, perm):
    def body(a_ref, h0_ref, g_ref, gb_ref, d_ref, p_ref, hh_ref, s_ref):
        deg = jnp.maximum(d_ref[0, :] + d_ref[1, :], 1.0)
        nd = lax.rsqrt(deg)
        hv = EPS * h0_ref[...] + (a_ref[0] + a_ref[1]) * nd[:, None]
        hhv = hv * nd[:, None]
        pidx = jnp.broadcast_to(p_ref[...], (BT, H))
        hh_ref[...] = jnp.take_along_axis(hhv, pidx, axis=1).astype(_BF16)
        sdot = lax.dot_general(g_ref[...], hv, _DN, precision=_PREC)
        ri = lax.broadcasted_iota(jnp.int32, (8, BT), 0)
        s_ref[...] = sdot + jnp.where(ri == 0, gb_ref[0, 0], 0.0)

    return pl.pallas_call(
        body,
        grid=(NPAD // BT,),
        in_specs=[
            pl.BlockSpec((2, BT, H), lambda i: (0, i, 0)),
            pl.BlockSpec((BT, H), lambda i: (i, 0)),
            pl.BlockSpec((8, H), lambda i: (0, 0)),
            pl.BlockSpec((1, 1), lambda i: (0, 0)),
            pl.BlockSpec((2, BT), lambda i: (0, i)),
            pl.BlockSpec((1, H), lambda i: (0, 0)),
        ],
        out_specs=[
            pl.BlockSpec((BT, H), lambda i: (i, 0)),
            pl.BlockSpec((8, BT), lambda i: (0, i)),
        ],
        out_shape=[
            jax.ShapeDtypeStruct((NPAD, H), _BF16),
            jax.ShapeDtypeStruct((8, NPAD), _f32),
        ],
    )(agg, h0, G8, gb, deg2, perm)


BF = 2000


def _tc_final(agg, h0, t2_W, t2_b2, deg2t):
    def body(a_ref, h0_ref, w_ref, b_ref, d_ref, o_ref):
        deg = jnp.maximum(d_ref[:, 0] + d_ref[:, 1], 1.0)
        nd = lax.rsqrt(deg)
        hv = EPS * h0_ref[...] + (a_ref[0] + a_ref[1]) * nd[:, None]
        o = lax.dot_general(hv, w_ref[...], _DN, precision=_PREC) + b_ref[...]
        m = jnp.max(o, axis=1, keepdims=True)
        z = o - m
        lse = jnp.log(jnp.sum(jnp.exp(z), axis=1, keepdims=True))
        o_ref[...] = z - lse

    return pl.pallas_call(
        body,
        grid=(N // BF,),
        in_specs=[
            pl.BlockSpec((2, BF, H), lambda i: (0, i, 0)),
            pl.BlockSpec((BF, H), lambda i: (i, 0)),
            pl.BlockSpec((C, H), lambda i: (0, 0)),
            pl.BlockSpec((1, C), lambda i: (0, 0)),
            pl.BlockSpec((BF, 2), lambda i: (i, 0)),
        ],
        out_specs=pl.BlockSpec((BF, C), lambda i: (i, 0)),
        out_shape=jax.ShapeDtypeStruct((N, C), _f32),
    )(agg, h0, t2_W, t2_b2, deg2t)


def kernel(x, edge_index, t1_W, t1_b, t2_W, t2_b, gate_W, gate_b):
    E = edge_index.shape[1]
    ept = -(-E // NW)
    ch = -(-ept // CHUNK)
    ch = -(-ch // 4) * 4
    EP = NW * ch * CHUNK

    row = jnp.pad(edge_index[0], (0, EP - E), constant_values=DUMMY)
    col = jnp.pad(edge_index[1], (0, EP - E), constant_values=DUMMY)
    row3d = row.reshape(NW, ch, CHUNK)
    col3d = col.reshape(NW, ch, CHUNK)

    x_pad = jnp.pad(x, ((0, NPAD - x.shape[0]), (0, 0)))
    t1_b2 = t1_b.reshape(1, H)
    t2_b2 = t2_b.reshape(1, C)
    G80 = jnp.zeros((8, H), _f32).at[0].set(gate_W[0, :H]).at[1].set(gate_W[0, H:])
    G81 = jnp.zeros((8, H), _f32).at[0].set(gate_W[1, :H]).at[1].set(gate_W[1, H:])
    gb0 = gate_b[0].reshape(1, 1)
    gb1 = gate_b[1].reshape(1, 1)
    zeros_big = jnp.zeros((NPAD, H), _f32)

    perm = jnp.asarray(_PERM).reshape(1, H)

    deg2 = _sc_deg(row3d, ch)
    h0, hh0, S0 = _tc_pre(x_pad, t1_W, t1_b2, G80, gb0, deg2, perm)
    agg = _sc_edge(hh0, S0, row3d, col3d, zeros_big, ch)
    hh1, S1 = _tc_mid(agg, h0, G81, gb1, deg2, perm)
    agg2 = _sc_edge(hh1, S1, row3d, col3d, zeros_big, ch)
    return _tc_final(agg2, h0, t2_W, t2_b2, deg2.T)


# speedup vs baseline: 2.3002x; 1.4305x over previous
import functools

import jax
import jax.numpy as jnp
import numpy as np
from jax import lax
from jax.experimental import pallas as pl
from jax.experimental.pallas import tpu as pltpu
from jax.experimental.pallas import tpu_sc as plsc

N = 10000
D = 128
H = 64
C = 16
EPS = 0.3

NPAD = 10240
DUMMY = NPAD - 1
NW = 32
CHUNK = 128
ROWS_PER_TILE = NPAD // NW * 2

_f32 = jnp.float32


def _mesh():
    return plsc.VectorSubcoreMesh(core_axis_name="c", subcore_axis_name="s")


_SC_PARAMS = pltpu.CompilerParams(
    needs_layout_passes=False, use_tc_tiling_on_sc=False)

_PERM = np.zeros((H,), np.int32)
for _q in range(H // 32):
    for _t in range(16):
        _PERM[32 * _q + 2 * _t] = 32 * _q + _t
        _PERM[32 * _q + 2 * _t + 1] = 32 * _q + 16 + _t
_BF16 = jnp.bfloat16


def _sc_deg(row3d, ch):
    @functools.partial(
        pl.kernel,
        out_type=jax.ShapeDtypeStruct((2, NPAD), _f32),
        mesh=_mesh(),
        compiler_params=_SC_PARAMS,
        scratch_types=[
            pltpu.VMEM((ch, CHUNK), jnp.int32),
            pltpu.VMEM((CHUNK,), _f32),
            pltpu.VMEM((ROWS_PER_TILE,), _f32),
            pltpu.VMEM_SHARED((NPAD,), _f32),
            pltpu.SemaphoreType.DMA,
        ],
    )
    def k(row_hbm, out_hbm, idx_v, ones_v, z_v, deg_sh, sem):
        c = lax.axis_index("c")
        s = lax.axis_index("s")
        wid = c * 16 + s
        for i in range(CHUNK // 16):
            ones_v[pl.ds(i * 16, 16)] = jnp.full((16,), 1.0, _f32)
        for i in range(ROWS_PER_TILE // 16):
            z_v[pl.ds(i * 16, 16)] = jnp.zeros((16,), _f32)
        pltpu.sync_copy(z_v, deg_sh.at[pl.ds(s * ROWS_PER_TILE, ROWS_PER_TILE)])
        pltpu.sync_copy(row_hbm.at[wid], idx_v)
        plsc.subcore_barrier()

        def body(j, carry):
            pltpu.sync_copy(ones_v, deg_sh.at[idx_v.at[j]], add=True)
            return carry

        lax.fori_loop(0, ch, body, 0)
        plsc.subcore_barrier()
        pltpu.sync_copy(
            deg_sh.at[pl.ds(s * ROWS_PER_TILE, ROWS_PER_TILE)],
            out_hbm.at[c, pl.ds(s * ROWS_PER_TILE, ROWS_PER_TILE)],
        )

    return k(row3d)


def _sc_edge(hh, S, row3d, col3d, zeros_big, ch):
    @functools.partial(
        pl.kernel,
        out_type=jax.ShapeDtypeStruct((2, NPAD, H), _f32),
        mesh=_mesh(),
        compiler_params=_SC_PARAMS,
        scratch_types=[
            pltpu.VMEM((ch, CHUNK), jnp.int32),
            pltpu.VMEM((ch, CHUNK), jnp.int32),
            pltpu.VMEM((NPAD,), _f32),
            pltpu.VMEM((NPAD,), _f32),
            pltpu.VMEM((CHUNK, H), _BF16),
            pltpu.VMEM((CHUNK, H), _BF16),
            pltpu.VMEM((CHUNK, H), _f32),
            pltpu.VMEM((CHUNK, H), _f32),
            pltpu.VMEM_SHARED((NPAD, H), _f32),
            pltpu.VMEM_SHARED((NPAD, H), _BF16),
            pltpu.SemaphoreType.DMA,
            pltpu.SemaphoreType.DMA,
            pltpu.SemaphoreType.DMA,
            pltpu.SemaphoreType.DMA,
        ],
    )
    def k(hh_hbm, s_hbm, row_hbm, col_hbm, z_hbm, out_hbm,
          rowi, coli, atab, btab, g0, g1, s0, s1, agg_sh, hh_sh,
          gsem0, gsem1, ssem0, ssem1):
        c = lax.axis_index("c")
        s = lax.axis_index("s")
        wid = c * 16 + s
        gbuf = (g0, g1)
        sbuf = (s0, s1)
        gsem = (gsem0, gsem1)
        ssem = (ssem0, ssem1)
        pltpu.sync_copy(s_hbm.at[0], atab)
        pltpu.sync_copy(s_hbm.at[1], btab)
        pltpu.sync_copy(row_hbm.at[wid], rowi)
        pltpu.sync_copy(col_hbm.at[wid], coli)
        pltpu.sync_copy(
            z_hbm.at[pl.ds(s * ROWS_PER_TILE, ROWS_PER_TILE)],
            agg_sh.at[pl.ds(s * ROWS_PER_TILE, ROWS_PER_TILE)],
        )
        pltpu.sync_copy(
            hh_hbm.at[pl.ds(s * ROWS_PER_TILE, ROWS_PER_TILE)],
            hh_sh.at[pl.ds(s * ROWS_PER_TILE, ROWS_PER_TILE)],
        )
        plsc.subcore_barrier()

        pltpu.async_copy(hh_sh.at[rowi.at[0]], g0, gsem0)

        def pair(jj, carry):
            for b in range(2):
                j = 2 * jj + b
                @pl.when(j + 1 < ch)
                def _():
                    pltpu.async_copy(
                        hh_sh.at[rowi.at[j + 1]], gbuf[1 - b], gsem[1 - b])
                pltpu.make_async_copy(
                    hh_sh.at[rowi.at[j]], gbuf[b], gsem[b]).wait()
                @pl.when(jj >= 1)
                def _():
                    pltpu.make_async_copy(
                        sbuf[b], agg_sh.at[coli.at[j]], ssem[b]).wait()
                for i in range(CHUNK // 16):
                    r16 = rowi[j, pl.ds(i * 16, 16)]
                    c16 = coli[j, pl.ds(i * 16, 16)]
                    ag = plsc.load_gather(atab, [r16])
                    bg = plsc.load_gather(btab, [c16])
                    t = ag + bg
                    sg = jnp.sign(t)
                    u = jnp.exp(-2.0 * jnp.abs(t))
                    nv = sg * (1.0 - u) / (1.0 + u)
                    for e in range(16):
                        be = jnp.take_along_axis(
                            nv, jnp.full((16,), e, jnp.int32), axis=0)
                        r = i * 16 + e
                        for q in range(H // 32):
                            x32 = gbuf[b][r, pl.ds(q * 32, 32)]
                            u, v = plsc.unpack(
                                x32, format=plsc.PackFormat.INTERLEAVED)
                            sbuf[b][r, pl.ds(q * 32, 16)] = u * be
                            sbuf[b][r, pl.ds(q * 32 + 16, 16)] = v * be
                pltpu.async_copy(
                    sbuf[b], agg_sh.at[coli.at[j]], ssem[b], add=True)
            return carry

        lax.fori_loop(0, ch // 2, pair, 0)
        for b in range(2):
            pltpu.make_async_copy(
                sbuf[b], agg_sh.at[coli.at[0]], ssem[b]).wait()
        plsc.subcore_barrier()
        pltpu.sync_copy(
            agg_sh.at[pl.ds(s * ROWS_PER_TILE, ROWS_PER_TILE)],
            out_hbm.at[c, pl.ds(s * ROWS_PER_TILE, ROWS_PER_TILE)],
        )

    return k(hh, S, row3d, col3d, zeros_big)


_DN = (((1,), (1,)), ((), ()))
_PREC = lax.Precision.HIGHEST
BT = 2048


def _tc_pre(x_pad, t1_W, t1_b2, G8, gb, deg2, perm):
    def body(x_ref, w_ref, b_ref, g_ref, gb_ref, d_ref, p_ref,
             h_ref, hh_ref, s_ref):
        xb = x_ref[...]
        hv = lax.dot_general(xb, w_ref[...], _DN, precision=_PREC) + b_ref[...]
        hv = jnp.maximum(hv, 0.0)
        h_ref[...] = hv
        deg = jnp.maximum(d_ref[0, :] + d_ref[1, :], 1.0)
        nd = lax.rsqrt(deg)
        hhv = hv * nd[:, None]
        pidx = jnp.broadcast_to(p_ref[...], (BT, H))
        hh_ref[...] = jnp.take_along_axis(hhv, pidx, axis=1).astype(_BF16)
        sdot = lax.dot_general(g_ref[...], hv, _DN, precision=_PREC)
        ri = lax.broadcasted_iota(jnp.int32, (8, BT), 0)
        s_ref[...] = sdot + jnp.where(ri == 0, gb_ref[0, 0], 0.0)

    return pl.pallas_call(
        body,
        grid=(NPAD // BT,),
        in_specs=[
            pl.BlockSpec((BT, D), lambda i: (i, 0)),
            pl.BlockSpec((H, D), lambda i: (0, 0)),
            pl.BlockSpec((1, H), lambda i: (0, 0)),
            pl.BlockSpec((8, H), lambda i: (0, 0)),
            pl.BlockSpec((1, 1), lambda i: (0, 0)),
            pl.BlockSpec((2, BT), lambda i: (0, i)),
            pl.BlockSpec((1, H), lambda i: (0, 0)),
        ],
        out_specs=[
            pl.BlockSpec((BT, H), lambda i: (i, 0)),
            pl.BlockSpec((BT, H), lambda i: (i, 0)),
            pl.BlockSpec((8, BT), lambda i: (0, i)),
        ],
        out_shape=[
            jax.ShapeDtypeStruct((NPAD, H), _f32),
            jax.ShapeDtypeStruct((NPAD, H), _BF16),
            jax.ShapeDtypeStruct((8, NPAD), _f32),
        ],
    )(x_pad, t1_W, t1_b2, G8, gb, deg2, perm)


def _tc_mid(agg, h0, G8, gb, deg2, perm):
    def body(a_ref, h0_ref, g_ref, gb_ref, d_ref, p_ref, hh_ref, s_ref):
        deg = jnp.maximum(d_ref[0, :] + d_ref[1, :], 1.0)
        nd = lax.rsqrt(deg)
        hv = EPS * h0_ref[...] + (a_ref[0] + a_ref[1]) * nd[:, None]
        hhv = hv * nd[:, None]
        pidx = jnp.broadcast_to(p_ref[...], (BT, H))
        hh_ref[...] = jnp.take_along_axis(hhv, pidx, axis=1).astype(_BF16)
        sdot = lax.dot_general(g_ref[...], hv, _DN, precision=_PREC)
        ri = lax.broadcasted_iota(jnp.int32, (8, BT), 0)
        s_ref[...] = sdot + jnp.where(ri == 0, gb_ref[0, 0], 0.0)

    return pl.pallas_call(
        body,
        grid=(NPAD // BT,),
        in_specs=[
            pl.BlockSpec((2, BT, H), lambda i: (0, i, 0)),
            pl.BlockSpec((BT, H), lambda i: (i, 0)),
            pl.BlockSpec((8, H), lambda i: (0, 0)),
            pl.BlockSpec((1, 1), lambda i: (0, 0)),
            pl.BlockSpec((2, BT), lambda i: (0, i)),
            pl.BlockSpec((1, H), lambda i: (0, 0)),
        ],
        out_specs=[
            pl.BlockSpec((BT, H), lambda i: (i, 0)),
            pl.BlockSpec((8, BT), lambda i: (0, i)),
        ],
        out_shape=[
            jax.ShapeDtypeStruct((NPAD, H), _BF16),
            jax.ShapeDtypeStruct((8, NPAD), _f32),
        ],
    )(agg, h0, G8, gb, deg2, perm)


BF = 2000


def _tc_final(agg, h0, t2_W, t2_b2, deg2t):
    def body(a_ref, h0_ref, w_ref, b_ref, d_ref, o_ref):
        deg = jnp.maximum(d_ref[:, 0] + d_ref[:, 1], 1.0)
        nd = lax.rsqrt(deg)
        hv = EPS * h0_ref[...] + (a_ref[0] + a_ref[1]) * nd[:, None]
        o = lax.dot_general(hv, w_ref[...], _DN, precision=_PREC) + b_ref[...]
        m = jnp.max(o, axis=1, keepdims=True)
        z = o - m
        lse = jnp.log(jnp.sum(jnp.exp(z), axis=1, keepdims=True))
        o_ref[...] = z - lse

    return pl.pallas_call(
        body,
        grid=(N // BF,),
        in_specs=[
            pl.BlockSpec((2, BF, H), lambda i: (0, i, 0)),
            pl.BlockSpec((BF, H), lambda i: (i, 0)),
            pl.BlockSpec((C, H), lambda i: (0, 0)),
            pl.BlockSpec((1, C), lambda i: (0, 0)),
            pl.BlockSpec((BF, 2), lambda i: (i, 0)),
        ],
        out_specs=pl.BlockSpec((BF, C), lambda i: (i, 0)),
        out_shape=jax.ShapeDtypeStruct((N, C), _f32),
    )(agg, h0, t2_W, t2_b2, deg2t)


def kernel(x, edge_index, t1_W, t1_b, t2_W, t2_b, gate_W, gate_b):
    E = edge_index.shape[1]
    ept = -(-E // NW)
    ch = -(-ept // CHUNK)
    ch = -(-ch // 4) * 4
    EP = NW * ch * CHUNK

    row = jnp.pad(edge_index[0], (0, EP - E), constant_values=DUMMY)
    col = jnp.pad(edge_index[1], (0, EP - E), constant_values=DUMMY)
    row3d = row.reshape(NW, ch, CHUNK)
    col3d = col.reshape(NW, ch, CHUNK)

    x_pad = jnp.pad(x, ((0, NPAD - x.shape[0]), (0, 0)))
    t1_b2 = t1_b.reshape(1, H)
    t2_b2 = t2_b.reshape(1, C)
    G80 = jnp.zeros((8, H), _f32).at[0].set(gate_W[0, :H]).at[1].set(gate_W[0, H:])
    G81 = jnp.zeros((8, H), _f32).at[0].set(gate_W[1, :H]).at[1].set(gate_W[1, H:])
    gb0 = gate_b[0].reshape(1, 1)
    gb1 = gate_b[1].reshape(1, 1)
    zeros_big = jnp.zeros((NPAD, H), _f32)

    perm = jnp.asarray(_PERM).reshape(1, H)

    deg2 = _sc_deg(row3d, ch)
    h0, hh0, S0 = _tc_pre(x_pad, t1_W, t1_b2, G80, gb0, deg2, perm)
    agg = _sc_edge(hh0, S0, row3d, col3d, zeros_big, ch)
    hh1, S1 = _tc_mid(agg, h0, G81, gb1, deg2, perm)
    agg2 = _sc_edge(hh1, S1, row3d, col3d, zeros_big, ch)
    return _tc_final(agg2, h0, t2_W, t2_b2, deg2.T)


# speedup vs baseline: 2.3311x; 1.0134x over previous
import functools

import jax
import jax.numpy as jnp
import numpy as np
from jax import lax
from jax.experimental import pallas as pl
from jax.experimental.pallas import tpu as pltpu
from jax.experimental.pallas import tpu_sc as plsc

N = 10000
D = 128
H = 64
C = 16
EPS = 0.3

NPAD = 10240
DUMMY = NPAD - 1
NW = 32
CHUNK = 128
ROWS_PER_TILE = NPAD // NW * 2

_f32 = jnp.float32


def _mesh():
    return plsc.VectorSubcoreMesh(core_axis_name="c", subcore_axis_name="s")


_SC_PARAMS = pltpu.CompilerParams(
    needs_layout_passes=False, use_tc_tiling_on_sc=False)

_PERM = np.zeros((H,), np.int32)
for _q in range(H // 32):
    for _t in range(16):
        _PERM[32 * _q + 2 * _t] = 32 * _q + _t
        _PERM[32 * _q + 2 * _t + 1] = 32 * _q + 16 + _t
_BF16 = jnp.bfloat16


def _sc_deg(row3d, ch):
    @functools.partial(
        pl.kernel,
        out_type=jax.ShapeDtypeStruct((2, NPAD), _f32),
        mesh=_mesh(),
        compiler_params=_SC_PARAMS,
        scratch_types=[
            pltpu.VMEM((ch, CHUNK), jnp.int32),
            pltpu.VMEM((CHUNK,), _f32),
            pltpu.VMEM((ROWS_PER_TILE,), _f32),
            pltpu.VMEM_SHARED((NPAD,), _f32),
            pltpu.SemaphoreType.DMA,
        ],
    )
    def k(row_hbm, out_hbm, idx_v, ones_v, z_v, deg_sh, sem):
        c = lax.axis_index("c")
        s = lax.axis_index("s")
        wid = c * 16 + s
        for i in range(CHUNK // 16):
            ones_v[pl.ds(i * 16, 16)] = jnp.full((16,), 1.0, _f32)
        for i in range(ROWS_PER_TILE // 16):
            z_v[pl.ds(i * 16, 16)] = jnp.zeros((16,), _f32)
        pltpu.sync_copy(z_v, deg_sh.at[pl.ds(s * ROWS_PER_TILE, ROWS_PER_TILE)])
        pltpu.sync_copy(row_hbm.at[wid], idx_v)
        plsc.subcore_barrier()

        def body(j, carry):
            pltpu.async_copy(ones_v, deg_sh.at[idx_v.at[j]], sem, add=True)
            return carry

        lax.fori_loop(0, ch, body, 0)

        def drain(j, carry):
            pltpu.make_async_copy(ones_v, deg_sh.at[idx_v.at[0]], sem).wait()
            return carry

        lax.fori_loop(0, ch, drain, 0)
        plsc.subcore_barrier()
        pltpu.sync_copy(
            deg_sh.at[pl.ds(s * ROWS_PER_TILE, ROWS_PER_TILE)],
            out_hbm.at[c, pl.ds(s * ROWS_PER_TILE, ROWS_PER_TILE)],
        )

    return k(row3d)


def _sc_edge(hh, S, row3d, col3d, ch):
    @functools.partial(
        pl.kernel,
        out_type=jax.ShapeDtypeStruct((2, NPAD, H), _f32),
        mesh=_mesh(),
        compiler_params=_SC_PARAMS,
        scratch_types=[
            pltpu.VMEM((ch, CHUNK), jnp.int32),
            pltpu.VMEM((ch, CHUNK), jnp.int32),
            pltpu.VMEM((NPAD,), _f32),
            pltpu.VMEM((NPAD,), _f32),
            pltpu.VMEM((CHUNK, H), _BF16),
            pltpu.VMEM((CHUNK, H), _BF16),
            pltpu.VMEM((CHUNK, H), _f32),
            pltpu.VMEM((CHUNK, H), _f32),
            pltpu.VMEM_SHARED((NPAD, H), _f32),
            pltpu.VMEM_SHARED((NPAD, H), _BF16),
            pltpu.SemaphoreType.DMA,
            pltpu.SemaphoreType.DMA,
            pltpu.SemaphoreType.DMA,
            pltpu.SemaphoreType.DMA,
        ],
    )
    def k(hh_hbm, s_hbm, row_hbm, col_hbm, out_hbm,
          rowi, coli, atab, btab, g0, g1, s0, s1, agg_sh, hh_sh,
          gsem0, gsem1, ssem0, ssem1):
        c = lax.axis_index("c")
        s = lax.axis_index("s")
        wid = c * 16 + s
        gbuf = (g0, g1)
        sbuf = (s0, s1)
        gsem = (gsem0, gsem1)
        ssem = (ssem0, ssem1)
        pltpu.sync_copy(s_hbm.at[0], atab)
        pltpu.sync_copy(s_hbm.at[1], btab)
        pltpu.sync_copy(row_hbm.at[wid], rowi)
        pltpu.sync_copy(col_hbm.at[wid], coli)
        def zrow(r, carry):
            for q in range(H // 16):
                s0[r, pl.ds(q * 16, 16)] = jnp.zeros((16,), _f32)
            return carry

        lax.fori_loop(0, CHUNK, zrow, 0)
        for z in range(ROWS_PER_TILE // CHUNK):
            pltpu.async_copy(
                s0, agg_sh.at[pl.ds(s * ROWS_PER_TILE + z * CHUNK, CHUNK)],
                ssem0)
        for z in range(ROWS_PER_TILE // CHUNK):
            pltpu.make_async_copy(
                s0, agg_sh.at[pl.ds(s * ROWS_PER_TILE, CHUNK)], ssem0).wait()
        pltpu.sync_copy(
            hh_hbm.at[pl.ds(s * ROWS_PER_TILE, ROWS_PER_TILE)],
            hh_sh.at[pl.ds(s * ROWS_PER_TILE, ROWS_PER_TILE)],
        )
        plsc.subcore_barrier()

        pltpu.async_copy(hh_sh.at[rowi.at[0]], g0, gsem0)

        def pair(jj, carry):
            for b in range(2):
                j = 2 * jj + b
                @pl.when(j + 1 < ch)
                def _():
                    pltpu.async_copy(
                        hh_sh.at[rowi.at[j + 1]], gbuf[1 - b], gsem[1 - b])
                pltpu.make_async_copy(
                    hh_sh.at[rowi.at[j]], gbuf[b], gsem[b]).wait()
                @pl.when(jj >= 1)
                def _():
                    pltpu.make_async_copy(
                        sbuf[b], agg_sh.at[coli.at[j]], ssem[b]).wait()
                for i in range(CHUNK // 16):
                    r16 = rowi[j, pl.ds(i * 16, 16)]
                    c16 = coli[j, pl.ds(i * 16, 16)]
                    ag = plsc.load_gather(atab, [r16])
                    bg = plsc.load_gather(btab, [c16])
                    t = ag + bg
                    sg = jnp.sign(t)
                    u = jnp.exp(-2.0 * jnp.abs(t))
                    nv = sg * (1.0 - u) / (1.0 + u)
                    for e in range(16):
                        be = jnp.take_along_axis(
                            nv, jnp.full((16,), e, jnp.int32), axis=0)
                        r = i * 16 + e
                        for q in range(H // 32):
                            x32 = gbuf[b][r, pl.ds(q * 32, 32)]
                            u, v = plsc.unpack(
                                x32, format=plsc.PackFormat.INTERLEAVED)
                            sbuf[b][r, pl.ds(q * 32, 16)] = u * be
                            sbuf[b][r, pl.ds(q * 32 + 16, 16)] = v * be
                pltpu.async_copy(
                    sbuf[b], agg_sh.at[coli.at[j]], ssem[b], add=True)
            return carry

        lax.fori_loop(0, ch // 2, pair, 0)
        for b in range(2):
            pltpu.make_async_copy(
                sbuf[b], agg_sh.at[coli.at[0]], ssem[b]).wait()
        plsc.subcore_barrier()
        pltpu.sync_copy(
            agg_sh.at[pl.ds(s * ROWS_PER_TILE, ROWS_PER_TILE)],
            out_hbm.at[c, pl.ds(s * ROWS_PER_TILE, ROWS_PER_TILE)],
        )

    return k(hh, S, row3d, col3d)


_DN = (((1,), (1,)), ((), ()))
_PREC = lax.Precision.HIGHEST
BT = 2048


def _tc_pre(x_pad, t1_W, t1_b2, G8, gb, deg2, perm):
    def body(x_ref, w_ref, b_ref, g_ref, gb_ref, d_ref, p_ref,
             h_ref, hh_ref, s_ref, nd_ref):
        xb = x_ref[...]
        hv = lax.dot_general(xb, w_ref[...], _DN, precision=_PREC) + b_ref[...]
        hv = jnp.maximum(hv, 0.0)
        h_ref[...] = hv
        deg = jnp.maximum(d_ref[0, :] + d_ref[1, :], 1.0)
        nd = lax.rsqrt(deg)
        hhv = hv * nd[:, None]
        pidx = jnp.broadcast_to(p_ref[...], (BT, H))
        hh_ref[...] = jnp.take_along_axis(hhv, pidx, axis=1).astype(_BF16)
        sdot = lax.dot_general(g_ref[...], hv, _DN, precision=_PREC)
        ri = lax.broadcasted_iota(jnp.int32, (8, BT), 0)
        s_ref[...] = sdot + jnp.where(ri == 0, gb_ref[0, 0], 0.0)
        ci = lax.broadcasted_iota(jnp.int32, (BT, 8), 1)
        nd_ref[...] = jnp.where(
            ci == 0, jnp.broadcast_to(nd[:, None], (BT, 8)), 0.0)

    return pl.pallas_call(
        body,
        grid=(NPAD // BT,),
        in_specs=[
            pl.BlockSpec((BT, D), lambda i: (i, 0)),
            pl.BlockSpec((H, D), lambda i: (0, 0)),
            pl.BlockSpec((1, H), lambda i: (0, 0)),
            pl.BlockSpec((8, H), lambda i: (0, 0)),
            pl.BlockSpec((1, 1), lambda i: (0, 0)),
            pl.BlockSpec((2, BT), lambda i: (0, i)),
            pl.BlockSpec((1, H), lambda i: (0, 0)),
        ],
        out_specs=[
            pl.BlockSpec((BT, H), lambda i: (i, 0)),
            pl.BlockSpec((BT, H), lambda i: (i, 0)),
            pl.BlockSpec((8, BT), lambda i: (0, i)),
            pl.BlockSpec((BT, 8), lambda i: (i, 0)),
        ],
        out_shape=[
            jax.ShapeDtypeStruct((NPAD, H), _f32),
            jax.ShapeDtypeStruct((NPAD, H), _BF16),
            jax.ShapeDtypeStruct((8, NPAD), _f32),
            jax.ShapeDtypeStruct((NPAD, 8), _f32),
        ],
    )(x_pad, t1_W, t1_b2, G8, gb, deg2, perm)


def _tc_mid(agg, h0, G8, gb, deg2, perm):
    def body(a_ref, h0_ref, g_ref, gb_ref, d_ref, p_ref, hh_ref, s_ref):
        deg = jnp.maximum(d_ref[0, :] + d_ref[1, :], 1.0)
        nd = lax.rsqrt(deg)
        hv = EPS * h0_ref[...] + (a_ref[0] + a_ref[1]) * nd[:, None]
        hhv = hv * nd[:, None]
        pidx = jnp.broadcast_to(p_ref[...], (BT, H))
        hh_ref[...] = jnp.take_along_axis(hhv, pidx, axis=1).astype(_BF16)
        sdot = lax.dot_general(g_ref[...], hv, _DN, precision=_PREC)
        ri = lax.broadcasted_iota(jnp.int32, (8, BT), 0)
        s_ref[...] = sdot + jnp.where(ri == 0, gb_ref[0, 0], 0.0)

    return pl.pallas_call(
        body,
        grid=(NPAD // BT,),
        in_specs=[
            pl.BlockSpec((2, BT, H), lambda i: (0, i, 0)),
            pl.BlockSpec((BT, H), lambda i: (i, 0)),
            pl.BlockSpec((8, H), lambda i: (0, 0)),
            pl.BlockSpec((1, 1), lambda i: (0, 0)),
            pl.BlockSpec((2, BT), lambda i: (0, i)),
            pl.BlockSpec((1, H), lambda i: (0, 0)),
        ],
        out_specs=[
            pl.BlockSpec((BT, H), lambda i: (i, 0)),
            pl.BlockSpec((8, BT), lambda i: (0, i)),
        ],
        out_shape=[
            jax.ShapeDtypeStruct((NPAD, H), _BF16),
            jax.ShapeDtypeStruct((8, NPAD), _f32),
        ],
    )(agg, h0, G8, gb, deg2, perm)


BF = 2000


def _tc_final(agg, h0, t2_W, t2_b2, ndpad):
    def body(a_ref, h0_ref, w_ref, b_ref, d_ref, o_ref):
        nd = d_ref[:, 0]
        hv = EPS * h0_ref[...] + (a_ref[0] + a_ref[1]) * nd[:, None]
        o = lax.dot_general(hv, w_ref[...], _DN, precision=_PREC) + b_ref[...]
        m = jnp.max(o, axis=1, keepdims=True)
        z = o - m
        lse = jnp.log(jnp.sum(jnp.exp(z), axis=1, keepdims=True))
        o_ref[...] = z - lse

    return pl.pallas_call(
        body,
        grid=(N // BF,),
        in_specs=[
            pl.BlockSpec((2, BF, H), lambda i: (0, i, 0)),
            pl.BlockSpec((BF, H), lambda i: (i, 0)),
            pl.BlockSpec((C, H), lambda i: (0, 0)),
            pl.BlockSpec((1, C), lambda i: (0, 0)),
            pl.BlockSpec((BF, 8), lambda i: (i, 0)),
        ],
        out_specs=pl.BlockSpec((BF, C), lambda i: (i, 0)),
        out_shape=jax.ShapeDtypeStruct((N, C), _f32),
    )(agg, h0, t2_W, t2_b2, ndpad)


def kernel(x, edge_index, t1_W, t1_b, t2_W, t2_b, gate_W, gate_b):
    E = edge_index.shape[1]
    ept = -(-E // NW)
    ch = -(-ept // CHUNK)
    ch = -(-ch // 4) * 4
    EP = NW * ch * CHUNK

    row = jnp.pad(edge_index[0], (0, EP - E), constant_values=DUMMY)
    col = jnp.pad(edge_index[1], (0, EP - E), constant_values=DUMMY)
    row3d = row.reshape(NW, ch, CHUNK)
    col3d = col.reshape(NW, ch, CHUNK)

    x_pad = jnp.pad(x, ((0, NPAD - x.shape[0]), (0, 0)))
    t1_b2 = t1_b.reshape(1, H)
    t2_b2 = t2_b.reshape(1, C)
    G80 = jnp.zeros((8, H), _f32).at[0].set(gate_W[0, :H]).at[1].set(gate_W[0, H:])
    G81 = jnp.zeros((8, H), _f32).at[0].set(gate_W[1, :H]).at[1].set(gate_W[1, H:])
    gb0 = gate_b[0].reshape(1, 1)
    gb1 = gate_b[1].reshape(1, 1)

    perm = jnp.asarray(_PERM).reshape(1, H)

    deg2 = _sc_deg(row3d, ch)
    h0, hh0, S0, ndpad = _tc_pre(x_pad, t1_W, t1_b2, G80, gb0, deg2, perm)
    agg = _sc_edge(hh0, S0, row3d, col3d, ch)
    hh1, S1 = _tc_mid(agg, h0, G81, gb1, deg2, perm)
    agg2 = _sc_edge(hh1, S1, row3d, col3d, ch)
    return _tc_final(agg2, h0, t2_W, t2_b2, ndpad)
